# baseline (device time: 21690 ns/iter reference)
import jax
import jax.numpy as jnp
from jax import lax
from jax.experimental import pallas as pl
from jax.experimental.pallas import tpu as pltpu

N_DEV = 32
T = 256
V_LOC = 4096
STAT_ROWS = 8


def kernel(x, W, labels):
    labels2d = labels.reshape(1, T)

    def body(x_ref, w_ref, lab_ref, out_ref,
             stats_ref, gather_ref, send_sems, recv_sems):
        my = lax.axis_index("i")

        logits = jnp.dot(
            x_ref[...].astype(jnp.bfloat16),
            w_ref[...].astype(jnp.bfloat16),
            preferred_element_type=jnp.float32,
        )
        m_loc = jnp.max(logits, axis=1)
        s_loc = jnp.sum(jnp.exp(logits - m_loc[:, None]), axis=1)

        lab = lab_ref[0, :][:, None]
        col = jax.lax.broadcasted_iota(jnp.int32, (T, V_LOC), 1) + my * V_LOC
        onehot = col == lab
        l_loc = jnp.sum(jnp.where(onehot, logits, 0.0), axis=1)

        stats_ref[0, :] = m_loc
        stats_ref[1, :] = s_loc
        stats_ref[2, :] = l_loc
        stats_ref[3:, :] = jnp.zeros((STAT_ROWS - 3, T), jnp.float32)
        gather_ref[my] = stats_ref[...]

        barrier_sem = pltpu.get_barrier_semaphore()
        for off in range(1, N_DEV):
            p = lax.rem(my + off, N_DEV)
            pl.semaphore_signal(
                barrier_sem, inc=1,
                device_id=(p,), device_id_type=pl.DeviceIdType.MESH,
            )
        pl.semaphore_wait(barrier_sem, N_DEV - 1)

        sends = []
        for off in range(1, N_DEV):
            p = lax.rem(my + off, N_DEV)
            rdma = pltpu.make_async_remote_copy(
                src_ref=stats_ref,
                dst_ref=gather_ref.at[my],
                send_sem=send_sems.at[off - 1],
                recv_sem=recv_sems.at[my],
                device_id=(p,),
                device_id_type=pl.DeviceIdType.MESH,
            )
            rdma.start()
            sends.append(rdma)

        for off in range(1, N_DEV):
            q = lax.rem(my + off, N_DEV)
            recv = pltpu.make_async_remote_copy(
                src_ref=stats_ref,
                dst_ref=gather_ref.at[q],
                send_sem=send_sems.at[N_DEV - 1],
                recv_sem=recv_sems.at[q],
                device_id=(q,),
                device_id_type=pl.DeviceIdType.MESH,
            )
            recv.wait_recv()

        g = gather_ref[...]
        m_k = g[:, 0, :]
        s_k = g[:, 1, :]
        l_k = g[:, 2, :]
        m = jnp.max(m_k, axis=0)
        s = jnp.sum(s_k * jnp.exp(m_k - m[None, :]), axis=0)
        nll = m + jnp.log(s) - jnp.sum(l_k, axis=0)
        out_ref[0, :] = nll

        for rdma in sends:
            rdma.wait_send()

    out = pl.pallas_call(
        body,
        out_shape=jax.ShapeDtypeStruct((1, T), jnp.float32),
        in_specs=[
            pl.BlockSpec(memory_space=pltpu.VMEM),
            pl.BlockSpec(memory_space=pltpu.VMEM),
            pl.BlockSpec(memory_space=pltpu.VMEM),
        ],
        out_specs=pl.BlockSpec(memory_space=pltpu.VMEM),
        scratch_shapes=[
            pltpu.VMEM((STAT_ROWS, T), jnp.float32),
            pltpu.VMEM((N_DEV, STAT_ROWS, T), jnp.float32),
            pltpu.SemaphoreType.DMA((N_DEV,)),
            pltpu.SemaphoreType.DMA((N_DEV,)),
        ],
        compiler_params=pltpu.CompilerParams(collective_id=0),
    )(x, W, labels2d)
    return out.reshape(T)


# device time: 18819 ns/iter; 1.1526x vs baseline; 1.1526x over previous
import jax
import jax.numpy as jnp
from jax import lax
from jax.experimental import pallas as pl
from jax.experimental.pallas import tpu as pltpu

N_DEV = 32
T = 256
V_LOC = 4096
STAT_ROWS = 4


def kernel(x, W, labels):
    labels2d = labels.reshape(1, T)

    def body(x_ref, w_ref, lab_ref, out_ref,
             stats_ref, gather_ref, send_sems, recv_sems):
        my = lax.axis_index("i")

        barrier_sem = pltpu.get_barrier_semaphore()
        for off in range(1, N_DEV):
            p = lax.rem(my + off, N_DEV)
            pl.semaphore_signal(
                barrier_sem, inc=1,
                device_id=(p,), device_id_type=pl.DeviceIdType.MESH,
            )

        logits = jnp.dot(
            x_ref[...].astype(jnp.bfloat16),
            w_ref[...].astype(jnp.bfloat16),
            preferred_element_type=jnp.float32,
        )
        m_loc = jnp.max(logits, axis=1)
        s_loc = jnp.sum(jnp.exp(logits - m_loc[:, None]), axis=1)

        lab = lab_ref[0, :][:, None]
        col = jax.lax.broadcasted_iota(jnp.int32, (T, V_LOC), 1) + my * V_LOC
        onehot = col == lab
        l_loc = jnp.sum(jnp.where(onehot, logits, 0.0), axis=1)

        stats_ref[0, :] = m_loc
        stats_ref[1, :] = s_loc
        stats_ref[2, :] = l_loc
        stats_ref[3:, :] = jnp.zeros((STAT_ROWS - 3, T), jnp.float32)
        gather_ref[my] = stats_ref[...]

        pl.semaphore_wait(barrier_sem, N_DEV - 1)

        sends = []
        for off in range(1, N_DEV):
            p = lax.rem(my + off, N_DEV)
            rdma = pltpu.make_async_remote_copy(
                src_ref=stats_ref,
                dst_ref=gather_ref.at[my],
                send_sem=send_sems.at[off - 1],
                recv_sem=recv_sems.at[my],
                device_id=(p,),
                device_id_type=pl.DeviceIdType.MESH,
            )
            rdma.start()
            sends.append(rdma)

        for off in range(1, N_DEV):
            q = lax.rem(my + off, N_DEV)
            recv = pltpu.make_async_remote_copy(
                src_ref=stats_ref,
                dst_ref=gather_ref.at[q],
                send_sem=send_sems.at[N_DEV - 1],
                recv_sem=recv_sems.at[q],
                device_id=(q,),
                device_id_type=pl.DeviceIdType.MESH,
            )
            recv.wait_recv()

        g = gather_ref[...]
        m_k = g[:, 0, :]
        s_k = g[:, 1, :]
        l_k = g[:, 2, :]
        m = jnp.max(m_k, axis=0)
        s = jnp.sum(s_k * jnp.exp(m_k - m[None, :]), axis=0)
        nll = m + jnp.log(s) - jnp.sum(l_k, axis=0)
        out_ref[0, :] = nll

        for rdma in sends:
            rdma.wait_send()

    out = pl.pallas_call(
        body,
        out_shape=jax.ShapeDtypeStruct((1, T), jnp.float32),
        in_specs=[
            pl.BlockSpec(memory_space=pltpu.VMEM),
            pl.BlockSpec(memory_space=pltpu.VMEM),
            pl.BlockSpec(memory_space=pltpu.VMEM),
        ],
        out_specs=pl.BlockSpec(memory_space=pltpu.VMEM),
        scratch_shapes=[
            pltpu.VMEM((STAT_ROWS, T), jnp.float32),
            pltpu.VMEM((N_DEV, STAT_ROWS, T), jnp.float32),
            pltpu.SemaphoreType.DMA((N_DEV,)),
            pltpu.SemaphoreType.DMA((N_DEV,)),
        ],
        compiler_params=pltpu.CompilerParams(collective_id=0),
    )(x, W, labels2d)
    return out.reshape(T)
